# baseline (speedup 1.0000x reference)
import jax
import jax.numpy as jnp
from jax import lax
from jax.experimental import pallas as pl
from jax.experimental.pallas import tpu as pltpu
from jax.experimental.pallas import tpu_sc as plsc

N_CLS = 100000
CTX_DIM = 512
B = 1024
NC, NS = 2, 16
NW = NC * NS
BPW = B // NW
CHUNK = 2
NCH = BPW // CHUNK
PADW = 4 * BPW


def _sc_body(ci_hbm, gi_hbm, hi_hbm, sgl_hbm, all_hbm, cls_hbm, gph_hbm,
             out_hbm, ci_v, gi_v, hi_v, asm0, asm1,
             sem_g0, sem_g1, sem_o0, sem_o1, sem_a):
    wid = lax.axis_index("s") * NC + lax.axis_index("c")
    base = wid * BPW
    asm = (asm0, asm1)
    sem_g = (sem_g0, sem_g1)
    sem_o = (sem_o0, sem_o1)

    pltpu.sync_copy(ci_hbm.at[pl.ds(wid * PADW, PADW)], ci_v)
    pltpu.sync_copy(gi_hbm.at[pl.ds(wid * PADW, PADW)], gi_v)
    pltpu.sync_copy(hi_hbm.at[pl.ds(wid * PADW, PADW)], hi_v)
    for bf in (0, 1):
        for e in range(CHUNK):
            pltpu.sync_copy(all_hbm.at[0], asm[bf].at[e, pl.ds(0, 16), :])

    def fire_gathers(j, bf):
        sl = pl.ds(j * 8, CHUNK)
        return [
            pltpu.async_copy(cls_hbm.at[gi_v.at[sl]],
                             asm[bf].at[:, pl.ds(16, 8), :], sem_g[bf]),
            pltpu.async_copy(gph_hbm.at[hi_v.at[sl]],
                             asm[bf].at[:, pl.ds(24, 4), :], sem_g[bf]),
            pltpu.async_copy(sgl_hbm.at[ci_v.at[sl]],
                             asm[bf].at[:, pl.ds(28, 4), :], sem_g[bf]),
        ]

    gd = {0: fire_gathers(0, 0)}
    outs = {0: [], 1: []}
    for j in range(NCH):
        bf = j % 2
        for d in gd[j]:
            d.wait()
        if j + 1 < NCH:
            nb = (j + 1) % 2
            for d in outs[nb]:
                d.wait()
            outs[nb] = []
            gd[j + 1] = fire_gathers(j + 1, nb)
        outs[bf] = [pltpu.async_copy(
            asm[bf], out_hbm.at[pl.ds(base + j * CHUNK, CHUNK), :, :],
            sem_o[bf])]
    for d in outs[0] + outs[1]:
        d.wait()


def kernel(cls_idx, cls_group_idx, graph_group_idx, ctx_single, ctx_all,
           ctx_cls, ctx_graph):
    mesh = plsc.VectorSubcoreMesh(core_axis_name="c", subcore_axis_name="s",
                                  num_cores=NC, num_subcores=NS)

    def repack(x):
        return jnp.repeat(x.reshape(-1, CHUNK), 4, axis=0).reshape(-1)

    run = pl.kernel(
        _sc_body,
        out_type=jax.ShapeDtypeStruct((B, 32, CTX_DIM), jnp.float32),
        mesh=mesh,
        compiler_params=pltpu.CompilerParams(use_tc_tiling_on_sc=True),
        scratch_types=[
            pltpu.VMEM((PADW,), jnp.int32),
            pltpu.VMEM((PADW,), jnp.int32),
            pltpu.VMEM((PADW,), jnp.int32),
            pltpu.VMEM((CHUNK, 32, CTX_DIM), jnp.float32),
            pltpu.VMEM((CHUNK, 32, CTX_DIM), jnp.float32),
            pltpu.SemaphoreType.DMA,
            pltpu.SemaphoreType.DMA,
            pltpu.SemaphoreType.DMA,
            pltpu.SemaphoreType.DMA,
            pltpu.SemaphoreType.DMA,
        ],
    )
    return run(repack(cls_idx), repack(cls_group_idx),
               repack(graph_group_idx), ctx_single, ctx_all, ctx_cls,
               ctx_graph)


# speedup vs baseline: 1.1444x; 1.1444x over previous
import jax
import jax.numpy as jnp
from jax import lax
from jax.experimental import pallas as pl
from jax.experimental.pallas import tpu as pltpu
from jax.experimental.pallas import tpu_sc as plsc

N_CLS = 100000
CTX_DIM = 512
B = 1024
NC, NS = 2, 16
NW = NC * NS
BPW = B // NW
CHUNK = 8
NCH = BPW // CHUNK
ALLR = 2


def _sc_body(ci_hbm, gi_hbm, hi_hbm, sgl_hbm, all_hbm, cls_hbm, gph_hbm,
             out_hbm, all_v, ci_v, gi_v, hi_v, sgl_v, gph_v, cls_v,
             sem_g, sem_o, sem_a):
    wid = lax.axis_index("s") * NC + lax.axis_index("c")
    base = wid * BPW

    for e in range(ALLR):
        pltpu.sync_copy(all_hbm.at[0], all_v.at[e])
    pltpu.sync_copy(ci_hbm.at[pl.ds(base, BPW)], ci_v)
    pltpu.sync_copy(gi_hbm.at[pl.ds(base, BPW)], gi_v)
    pltpu.sync_copy(hi_hbm.at[pl.ds(base, BPW)], hi_v)

    for j in range(NCH):
        sl = pl.ds(j * CHUNK, CHUNK)
        r0 = base + j * CHUNK
        g1 = pltpu.async_copy(sgl_hbm.at[ci_v.at[sl]], sgl_v, sem_g)
        g2 = pltpu.async_copy(cls_hbm.at[gi_v.at[sl]], cls_v, sem_g)
        g3 = pltpu.async_copy(gph_hbm.at[hi_v.at[sl]], gph_v, sem_g)
        outs = [pltpu.async_copy(
            all_v, out_hbm.at[pl.ds(r0 + k * ALLR, ALLR), pl.ds(0, 16), :],
            sem_a) for k in range(CHUNK // ALLR)]
        g1.wait()
        g2.wait()
        g3.wait()
        outs.append(pltpu.async_copy(
            cls_v, out_hbm.at[pl.ds(r0, CHUNK), pl.ds(16, 8), :], sem_o))
        outs.append(pltpu.async_copy(
            gph_v, out_hbm.at[pl.ds(r0, CHUNK), pl.ds(24, 4), :], sem_o))
        outs.append(pltpu.async_copy(
            sgl_v, out_hbm.at[pl.ds(r0, CHUNK), pl.ds(28, 4), :], sem_o))
        for d in outs:
            d.wait()


def kernel(cls_idx, cls_group_idx, graph_group_idx, ctx_single, ctx_all,
           ctx_cls, ctx_graph):
    mesh = plsc.VectorSubcoreMesh(core_axis_name="c", subcore_axis_name="s",
                                  num_cores=NC, num_subcores=NS)
    run = pl.kernel(
        _sc_body,
        out_type=jax.ShapeDtypeStruct((B, 32, CTX_DIM), jnp.float32),
        mesh=mesh,
        compiler_params=pltpu.CompilerParams(use_tc_tiling_on_sc=True),
        scratch_types=[
            pltpu.VMEM((ALLR, 16, CTX_DIM), jnp.float32),
            pltpu.VMEM((BPW,), jnp.int32),
            pltpu.VMEM((BPW,), jnp.int32),
            pltpu.VMEM((BPW,), jnp.int32),
            pltpu.VMEM((CHUNK, 4, CTX_DIM), jnp.float32),
            pltpu.VMEM((CHUNK, 4, CTX_DIM), jnp.float32),
            pltpu.VMEM((CHUNK, 8, CTX_DIM), jnp.float32),
            pltpu.SemaphoreType.DMA,
            pltpu.SemaphoreType.DMA,
            pltpu.SemaphoreType.DMA,
        ],
    )
    return run(cls_idx, cls_group_idx, graph_group_idx, ctx_single, ctx_all,
               ctx_cls, ctx_graph)


# speedup vs baseline: 1.3111x; 1.1457x over previous
import jax
import jax.numpy as jnp
from jax import lax
from jax.experimental import pallas as pl
from jax.experimental.pallas import tpu as pltpu
from jax.experimental.pallas import tpu_sc as plsc

N_CLS = 100000
N_CO_GRAPH = 9
CTX_DIM = 512
B = 1024
NC, NS = 2, 16
NW = NC * NS
BPW = B // NW
CHUNK = 8
NCH = BPW // CHUNK
ALLR = 2


def _sc_body(ci_hbm, gi_hbm, hi_hbm, sgl_hbm, all_hbm, cls_hbm, gph_hbm,
             out_hbm, all_v, ci_v, gi_v, hi_v, gphtab_v, sgl_v, cls_v,
             sem_g, sem_o, sem_a):
    wid = lax.axis_index("s") * NC + lax.axis_index("c")
    base = wid * BPW

    for e in range(ALLR):
        pltpu.sync_copy(all_hbm.at[0], all_v.at[e])
    pltpu.sync_copy(gph_hbm, gphtab_v)
    pltpu.sync_copy(ci_hbm.at[pl.ds(base, BPW)], ci_v)
    pltpu.sync_copy(gi_hbm.at[pl.ds(base, BPW)], gi_v)
    pltpu.sync_copy(hi_hbm.at[pl.ds(base, BPW)], hi_v)

    lanes = lax.iota(jnp.int32, 16)

    def scalar_at(vref, k):
        vec = vref[pl.ds((k // 16) * 16, 16)]
        return jnp.sum(jnp.where(lanes == (k % 16), vec, 0))

    for j in range(NCH):
        sl = pl.ds(j * CHUNK, CHUNK)
        r0 = base + j * CHUNK
        g1 = pltpu.async_copy(sgl_hbm.at[ci_v.at[sl]], sgl_v, sem_g)
        g2 = pltpu.async_copy(cls_hbm.at[gi_v.at[sl]], cls_v, sem_g)
        outs = [pltpu.async_copy(
            all_v, out_hbm.at[pl.ds(r0 + k * ALLR, ALLR), pl.ds(0, 16), :],
            sem_a) for k in range(CHUNK // ALLR)]
        for e in range(CHUNK):
            g = scalar_at(hi_v, j * CHUNK + e)
            outs.append(pltpu.async_copy(
                gphtab_v.at[g],
                out_hbm.at[r0 + e, pl.ds(24, 4), :], sem_a))
        g1.wait()
        g2.wait()
        outs.append(pltpu.async_copy(
            cls_v, out_hbm.at[pl.ds(r0, CHUNK), pl.ds(16, 8), :], sem_o))
        outs.append(pltpu.async_copy(
            sgl_v, out_hbm.at[pl.ds(r0, CHUNK), pl.ds(28, 4), :], sem_o))
        for d in outs:
            d.wait()


def kernel(cls_idx, cls_group_idx, graph_group_idx, ctx_single, ctx_all,
           ctx_cls, ctx_graph):
    mesh = plsc.VectorSubcoreMesh(core_axis_name="c", subcore_axis_name="s",
                                  num_cores=NC, num_subcores=NS)
    run = pl.kernel(
        _sc_body,
        out_type=jax.ShapeDtypeStruct((B, 32, CTX_DIM), jnp.float32),
        mesh=mesh,
        compiler_params=pltpu.CompilerParams(use_tc_tiling_on_sc=True,
                                             needs_layout_passes=False),
        scratch_types=[
            pltpu.VMEM((ALLR, 16, CTX_DIM), jnp.float32),
            pltpu.VMEM((BPW,), jnp.int32),
            pltpu.VMEM((BPW,), jnp.int32),
            pltpu.VMEM((BPW,), jnp.int32),
            pltpu.VMEM((N_CO_GRAPH, 4, CTX_DIM), jnp.float32),
            pltpu.VMEM((CHUNK, 4, CTX_DIM), jnp.float32),
            pltpu.VMEM((CHUNK, 8, CTX_DIM), jnp.float32),
            pltpu.SemaphoreType.DMA,
            pltpu.SemaphoreType.DMA,
            pltpu.SemaphoreType.DMA,
        ],
    )
    return run(cls_idx, cls_group_idx, graph_group_idx, ctx_single, ctx_all,
               ctx_cls, ctx_graph)


# speedup vs baseline: 1.5775x; 1.2032x over previous
import jax
import jax.numpy as jnp
from jax import lax
from jax.experimental import pallas as pl
from jax.experimental.pallas import tpu as pltpu
from jax.experimental.pallas import tpu_sc as plsc

N_CLS = 100000
N_CO_CLS = 20
N_CO_GRAPH = 9
CTX_DIM = 512
B = 1024
NC, NS = 2, 16
NW = NC * NS
BPW = B // NW
CHUNK = 8
NCH = BPW // CHUNK
ALLR = 2


def _sc_body(ci_hbm, gi_hbm, hi_hbm, sgl_hbm, all_hbm, cls_hbm, gph_hbm,
             out_hbm, all_sh, clstab_sh, ci_v, gi_v, hi_v, gphtab_v, sgl_v,
             sem_g, sem_o, sem_a):
    sid = lax.axis_index("s")
    wid = sid * NC + lax.axis_index("c")
    base = wid * BPW

    @pl.when(sid == 0)
    def _stage_shared():
        for e in range(ALLR):
            pltpu.sync_copy(all_hbm.at[0], all_sh.at[e])
        pltpu.sync_copy(cls_hbm, clstab_sh)
    pltpu.sync_copy(gph_hbm, gphtab_v)
    pltpu.sync_copy(ci_hbm.at[pl.ds(base, BPW)], ci_v)
    pltpu.sync_copy(gi_hbm.at[pl.ds(base, BPW)], gi_v)
    pltpu.sync_copy(hi_hbm.at[pl.ds(base, BPW)], hi_v)
    plsc.subcore_barrier()

    lanes = lax.iota(jnp.int32, 16)

    def scalar_at(vref, k):
        vec = vref[pl.ds((k // 16) * 16, 16)]
        return jnp.sum(jnp.where(lanes == (k % 16), vec, 0))

    for j in range(NCH):
        sl = pl.ds(j * CHUNK, CHUNK)
        r0 = base + j * CHUNK
        g1 = pltpu.async_copy(sgl_hbm.at[ci_v.at[sl]], sgl_v, sem_g)
        outs = [pltpu.async_copy(
            all_sh, out_hbm.at[pl.ds(r0 + k * ALLR, ALLR), pl.ds(0, 16), :],
            sem_a) for k in range(CHUNK // ALLR)]
        for e in range(CHUNK):
            c = scalar_at(gi_v, j * CHUNK + e)
            outs.append(pltpu.async_copy(
                clstab_sh.at[c],
                out_hbm.at[r0 + e, pl.ds(16, 8), :], sem_o))
            g = scalar_at(hi_v, j * CHUNK + e)
            outs.append(pltpu.async_copy(
                gphtab_v.at[g],
                out_hbm.at[r0 + e, pl.ds(24, 4), :], sem_a))
        g1.wait()
        outs.append(pltpu.async_copy(
            sgl_v, out_hbm.at[pl.ds(r0, CHUNK), pl.ds(28, 4), :], sem_o))
        for d in outs:
            d.wait()


def kernel(cls_idx, cls_group_idx, graph_group_idx, ctx_single, ctx_all,
           ctx_cls, ctx_graph):
    mesh = plsc.VectorSubcoreMesh(core_axis_name="c", subcore_axis_name="s",
                                  num_cores=NC, num_subcores=NS)
    run = pl.kernel(
        _sc_body,
        out_type=jax.ShapeDtypeStruct((B, 32, CTX_DIM), jnp.float32),
        mesh=mesh,
        compiler_params=pltpu.CompilerParams(use_tc_tiling_on_sc=True,
                                             needs_layout_passes=False),
        scratch_types=[
            pltpu.VMEM_SHARED((ALLR, 16, CTX_DIM), jnp.float32),
            pltpu.VMEM_SHARED((N_CO_CLS, 8, CTX_DIM), jnp.float32),
            pltpu.VMEM((BPW,), jnp.int32),
            pltpu.VMEM((BPW,), jnp.int32),
            pltpu.VMEM((BPW,), jnp.int32),
            pltpu.VMEM((N_CO_GRAPH, 4, CTX_DIM), jnp.float32),
            pltpu.VMEM((CHUNK, 4, CTX_DIM), jnp.float32),
            pltpu.SemaphoreType.DMA,
            pltpu.SemaphoreType.DMA,
            pltpu.SemaphoreType.DMA,
        ],
    )
    return run(cls_idx, cls_group_idx, graph_group_idx, ctx_single, ctx_all,
               ctx_cls, ctx_graph)
